# baseline (speedup 1.0000x reference)
import jax
import jax.numpy as jnp
from jax import lax
from jax.experimental import pallas as pl
from jax.experimental.pallas import tpu as pltpu
from jax.experimental.pallas import tpu_sc as plsc

_INFO = plsc.get_sparse_core_info()
_NC = _INFO.num_cores
_NS = _INFO.num_subcores
_L = _INFO.num_lanes
_NW = _NC * _NS

_BATCH = 16384
_D = 32
_BPW = _BATCH // _NW
_CH = 256
_NCH = _BPW // _CH


def _collab_body(users_hbm, movies_hbm, eu_hbm, em_hbm, bu_hbm, bm_hbm,
                 out_hbm, idx_u, idx_m, idx4_u, idx4_m, rows_u, rows_m,
                 bu_v, bm_v, out_v, sem):
    wid = lax.axis_index("s") * _NC + lax.axis_index("c")
    base = wid * _BPW

    pltpu.sync_copy(users_hbm.at[pl.ds(base, _BPW)], idx_u)
    pltpu.sync_copy(movies_hbm.at[pl.ds(base, _BPW)], idx_m)

    cpb1 = pltpu.async_copy(bu_hbm.at[idx_u], bu_v, sem)
    cpb2 = pltpu.async_copy(bm_hbm.at[idx_m], bm_v, sem)

    for v in range(_BPW // _L):
        idx4_u[pl.ds(v * _L, _L)] = idx_u[pl.ds(v * _L, _L)] >> 2
        idx4_m[pl.ds(v * _L, _L)] = idx_m[pl.ds(v * _L, _L)] >> 2

    cpb1.wait()
    cpb2.wait()

    lanes = lax.iota(jnp.int32, _L)
    onehots = [lanes == k for k in range(_L)]
    shuf8 = (lanes + 8) % _L
    shuf4 = (lanes + 4) % _L
    shuf2 = (lanes + 2) % _L
    shuf1 = (lanes + 1) % _L

    dnums = lax.GatherDimensionNumbers(
        offset_dims=(), collapsed_slice_dims=(0,), start_index_map=(0,))

    def shuffle(t, idx):
        return lax.gather(t, idx[:, None], dnums, slice_sizes=(1,),
                          mode=lax.GatherScatterMode.PROMISE_IN_BOUNDS)

    for s in range(_NCH):
        cp1 = pltpu.async_copy(
            eu_hbm.at[idx4_u.at[pl.ds(s * _CH, _CH)]], rows_u, sem)
        cp2 = pltpu.async_copy(
            em_hbm.at[idx4_m.at[pl.ds(s * _CH, _CH)]], rows_m, sem)
        cp1.wait()
        cp2.wait()

        def chunk(c, carry):
            b = c * _L
            dot = bu_v[pl.ds(s * _CH + b, _L)] + bm_v[pl.ds(s * _CH + b, _L)]
            ouv = (idx_u[pl.ds(s * _CH + b, _L)] & 3) * _D
            omv = (idx_m[pl.ds(s * _CH + b, _L)] & 3) * _D
            for k in range(_L):
                r = b + k
                ou = ouv[k]
                om = omv[k]
                t = (rows_u[r, pl.ds(ou, _L)] * rows_m[r, pl.ds(om, _L)] +
                     rows_u[r, pl.ds(ou + _L, _L)] *
                     rows_m[r, pl.ds(om + _L, _L)])
                t = t + shuffle(t, shuf8)
                t = t + shuffle(t, shuf4)
                t = t + shuffle(t, shuf2)
                t = t + shuffle(t, shuf1)
                dot = dot + jnp.where(onehots[k], t, 0.0)
            out_v[pl.ds(s * _CH + b, _L)] = 5.25 / (1.0 + jnp.exp(-dot))
            return carry

        lax.fori_loop(0, _CH // _L, chunk, 0)

    pltpu.sync_copy(out_v, out_hbm.at[pl.ds(base, _BPW)])


def kernel(users, movies, embedding_user, embedding_movie, bias_user,
           bias_movie):
    mesh = plsc.VectorSubcoreMesh(core_axis_name="c", subcore_axis_name="s")
    run = pl.kernel(
        _collab_body,
        mesh=mesh,
        compiler_params=pltpu.CompilerParams(use_tc_tiling_on_sc=False),
        out_type=jax.ShapeDtypeStruct((_BATCH,), jnp.float32),
        scratch_types=[
            pltpu.VMEM((_BPW,), jnp.int32),
            pltpu.VMEM((_BPW,), jnp.int32),
            pltpu.VMEM((_BPW,), jnp.int32),
            pltpu.VMEM((_BPW,), jnp.int32),
            pltpu.VMEM((_CH, 4 * _D), jnp.float32),
            pltpu.VMEM((_CH, 4 * _D), jnp.float32),
            pltpu.VMEM((_BPW,), jnp.float32),
            pltpu.VMEM((_BPW,), jnp.float32),
            pltpu.VMEM((_BPW,), jnp.float32),
            pltpu.SemaphoreType.DMA,
        ],
    )
    eu4 = embedding_user.reshape(-1, 4 * _D)
    em4 = embedding_movie.reshape(-1, 4 * _D)
    return run(users.astype(jnp.int32), movies.astype(jnp.int32),
               eu4, em4, bias_user, bias_movie)
